# baseline (speedup 1.0000x reference)
import functools

import jax
import jax.numpy as jnp
from jax import lax
from jax.experimental import pallas as pl
from jax.experimental.pallas import tpu as pltpu
from jax.experimental.pallas import tpu_sc as plsc

_N = 10000
_E = 320000
_D = 128
_DE = 16
_L = 3
_G = 64

_NC = 2
_NS = 16
_NW = _NC * _NS
_EPT = _E // _NW
_CH = 40
_NCH = _EPT // _CH
_BLK = 10
_NB = _NCH // _BLK
_NP = 10240
_NPT = _NP // _NS
_ZR = 80

_BE = 3200


def _embed_body(attr_ref, w_ref, b_ref, out_ref):
    a = attr_ref[...]
    out_ref[...] = (
        jnp.dot(a, w_ref[...], preferred_element_type=jnp.float32) + b_ref[...]
    )


_embed = pl.pallas_call(
    _embed_body,
    grid=(_E // _BE,),
    in_specs=[
        pl.BlockSpec((_BE, _DE), lambda i: (i, 0)),
        pl.BlockSpec((_DE, _D), lambda i: (0, 0)),
        pl.BlockSpec((1, _D), lambda i: (0, 0)),
    ],
    out_specs=pl.BlockSpec((_BE, _D), lambda i: (i, 0)),
    out_shape=jax.ShapeDtypeStruct((_E, _D), jnp.float32),
)


def _msgpass_body(h_hbm, ea_hbm, idx_hbm, out_hbm,
                  idxb, msgr, hr, acc, easem, gsem, ssem, isem):
    cid = lax.axis_index("c")
    sid = lax.axis_index("s")
    wid = cid * _NS + sid

    zero16 = jnp.zeros((16,), jnp.float32)

    def _zrow(i, c):
        for j in range(8):
            msgr[i, pl.ds(j * 16, 16)] = zero16
        return c

    lax.fori_loop(0, _ZR, _zrow, 0)
    for t in range(_NPT // _ZR):
        pltpu.sync_copy(msgr.at[pl.ds(0, _ZR)],
                        acc.at[pl.ds(sid * _NPT + t * _ZR, _ZR)])
    plsc.subcore_barrier()

    gbase = wid * _EPT

    def _fire(base, k2, p):
        b = k2 % 2
        pltpu.async_copy(ea_hbm.at[pl.ds(base + k2 * _CH, _CH)],
                         msgr.at[pl.ds(b * _CH, _CH)], easem.at[b])
        pltpu.async_copy(h_hbm.at[idxb.at[p, 2 * k2]],
                         hr.at[pl.ds(b * _CH, _CH)], gsem.at[b])

    def _wait_streams(base, k2, p):
        b = k2 % 2
        pltpu.make_async_copy(ea_hbm.at[pl.ds(base + k2 * _CH, _CH)],
                              msgr.at[pl.ds(b * _CH, _CH)],
                              easem.at[b]).wait()
        pltpu.make_async_copy(h_hbm.at[idxb.at[p, 2 * k2]],
                              hr.at[pl.ds(b * _CH, _CH)], gsem.at[b]).wait()

    def _fire_scatter(k2, p):
        b = k2 % 2
        pltpu.async_copy(hr.at[pl.ds(b * _CH, _CH)],
                         acc.at[idxb.at[p, 2 * k2 + 1]], ssem.at[b],
                         add=True)

    def _wait_scatter(b):
        pltpu.make_async_copy(hr.at[pl.ds(b * _CH, _CH)],
                              acc.at[idxb.at[0, 1]], ssem.at[b]).wait()

    def _relu(b):
        def _body(e, c):
            r = b * _CH + e
            for j in range(8):
                sl = pl.ds(j * 16, 16)
                hr[r, sl] = jnp.maximum(msgr[r, sl] + hr[r, sl], 0.0)
            return c
        lax.fori_loop(0, _CH, _body, 0)

    pltpu.sync_copy(idx_hbm.at[wid, 0], idxb.at[0])
    _fire(gbase, 0, 0)

    def _block(bb, c):
        p = lax.rem(bb, 2)
        pn = lax.rem(bb + 1, 2)
        base = gbase + bb * (_BLK * _CH)
        for k2 in range(_BLK):
            b = k2 % 2
            bn = (k2 + 1) % 2
            if k2 == 0:
                @pl.when(bb > 0)
                def _():
                    _wait_scatter(bn)
            else:
                _wait_scatter(bn)
            if k2 == 1:
                @pl.when(bb < _NB - 1)
                def _():
                    pltpu.async_copy(idx_hbm.at[wid, bb + 1], idxb.at[pn],
                                     isem)
            if k2 < _BLK - 1:
                _fire(base, k2 + 1, p)
            else:
                @pl.when(bb < _NB - 1)
                def _():
                    pltpu.make_async_copy(idx_hbm.at[wid, bb + 1],
                                          idxb.at[pn], isem).wait()
                    _fire(base + _BLK * _CH, 0, pn)
            _wait_streams(base, k2, p)
            _relu(b)
            _fire_scatter(k2, p)
        return c

    lax.fori_loop(0, _NB, _block, 0)
    _wait_scatter((_NCH - 1) % 2)
    plsc.subcore_barrier()

    for t in range(_NPT // _ZR):
        sl = pl.ds(sid * _NPT + t * _ZR, _ZR)
        pltpu.sync_copy(acc.at[sl], out_hbm.at[cid, sl])


@functools.cache
def _make_msgpass():
    return functools.partial(
        pl.kernel,
        mesh=plsc.VectorSubcoreMesh(core_axis_name="c", subcore_axis_name="s",
                                    num_cores=_NC, num_subcores=_NS),
        out_type=jax.ShapeDtypeStruct((_NC, _NP, _D), jnp.float32),
        scratch_types=[
            pltpu.VMEM((2, 2 * _BLK, _CH), jnp.int32),
            pltpu.VMEM((2 * _CH, _D), jnp.float32),
            pltpu.VMEM((2 * _CH, _D), jnp.float32),
            pltpu.VMEM_SHARED((_NP, _D), jnp.float32),
            pltpu.SemaphoreType.DMA((2,)),
            pltpu.SemaphoreType.DMA((2,)),
            pltpu.SemaphoreType.DMA((2,)),
            pltpu.SemaphoreType.DMA,
        ],
    )(_msgpass_body)


def _mlp_body(h_ref, p_ref, w1_ref, b1_ref, g_ref, be_ref, w2_ref, b2_ref,
              out_ref):
    z = h_ref[...] + p_ref[0, :_N] + p_ref[1, :_N]
    z = jnp.dot(z, w1_ref[...], preferred_element_type=jnp.float32) + b1_ref[...]
    mu = jnp.mean(z, axis=0, keepdims=True)
    var = jnp.mean((z - mu) * (z - mu), axis=0, keepdims=True)
    z = (z - mu) / jnp.sqrt(var + 1e-5) * g_ref[...] + be_ref[...]
    z = jnp.maximum(z, 0.0)
    z = jnp.dot(z, w2_ref[...], preferred_element_type=jnp.float32) + b2_ref[...]
    out_ref[...] = jnp.maximum(z, 0.0)


_mlp = pl.pallas_call(
    _mlp_body,
    out_shape=jax.ShapeDtypeStruct((_N, _D), jnp.float32),
)


def _pool_body(h_ref, b_ref, wo1_ref, bo1_ref, wo2_ref, bo2_ref, out_ref):
    bids = b_ref[...]
    gi = lax.broadcasted_iota(jnp.int32, (_N, _G), 1)
    mask = (bids == gi).astype(jnp.float32)
    dn = (((0,), (0,)), ((), ()))
    sums = lax.dot_general(mask, h_ref[...], dn,
                           preferred_element_type=jnp.float32)
    ones = jnp.ones((_N, 1), jnp.float32)
    cnt = lax.dot_general(mask, ones, dn,
                          preferred_element_type=jnp.float32)
    pooled = sums / jnp.maximum(cnt, 1.0)
    t = jnp.maximum(
        jnp.dot(pooled, wo1_ref[...], preferred_element_type=jnp.float32)
        + bo1_ref[...], 0.0)
    out_ref[...] = (
        jnp.dot(t, wo2_ref[...], preferred_element_type=jnp.float32)
        + bo2_ref[...])


_pool = pl.pallas_call(
    _pool_body,
    out_shape=jax.ShapeDtypeStruct((_G, _D), jnp.float32),
)


def kernel(x, edge_index, edge_attr, batch, edge_W, edge_b, W1, b1, gamma,
           beta, W2, b2, Wo1, bo1, Wo2, bo2):
    src = edge_index[0].reshape(_NW, _NCH, 1, _CH)
    dst = edge_index[1].reshape(_NW, _NCH, 1, _CH)
    idx = jnp.concatenate([src, dst], axis=2).reshape(_NW, _NB, 2 * _BLK, _CH)

    ea = _embed(edge_attr, edge_W[0], edge_b[0].reshape(1, _D))
    h = x
    for l in range(_L):
        parts = _make_msgpass()(h, ea, idx)
        if l + 1 < _L:
            ea = _embed(edge_attr, edge_W[l + 1], edge_b[l + 1].reshape(1, _D))
        h = _mlp(h, parts, W1[l], b1[l].reshape(1, _D),
                 gamma[l].reshape(1, _D), beta[l].reshape(1, _D),
                 W2[l], b2[l].reshape(1, _D))

    return _pool(h, batch.reshape(_N, 1), Wo1, bo1.reshape(1, _D),
                 Wo2, bo2.reshape(1, _D))


# speedup vs baseline: 1.1305x; 1.1305x over previous
import functools

import jax
import jax.numpy as jnp
from jax import lax
from jax.experimental import pallas as pl
from jax.experimental.pallas import tpu as pltpu
from jax.experimental.pallas import tpu_sc as plsc

_N = 10000
_E = 320000
_D = 128
_DE = 16
_L = 3
_G = 64

_NC = 2
_NS = 16
_NW = _NC * _NS
_EPT = _E // _NW
_CH = 80
_NCH = _EPT // _CH
_BLK = 10
_NBF = 12
_TAIL = _NCH - _NBF * _BLK
_NP = 10240
_NPT = _NP // _NS
_ZR = 160

_BE = 3200


def _embed_body(attr_ref, w_ref, b_ref, out_ref):
    a = attr_ref[...]
    for l in range(_L):
        out_ref[l] = (
            jnp.dot(a, w_ref[l], preferred_element_type=jnp.float32) + b_ref[l]
        )


_embed = pl.pallas_call(
    _embed_body,
    grid=(_E // _BE,),
    in_specs=[
        pl.BlockSpec((_BE, _DE), lambda i: (i, 0)),
        pl.BlockSpec((_L, _DE, _D), lambda i: (0, 0, 0)),
        pl.BlockSpec((_L, 1, _D), lambda i: (0, 0, 0)),
    ],
    out_specs=pl.BlockSpec((_L, _BE, _D), lambda i: (0, i, 0)),
    out_shape=jax.ShapeDtypeStruct((_L, _E, _D), jnp.float32),
)


def _msgpass_body(l, h_hbm, ea_hbm, idx_hbm, out_hbm,
                  idxb, msgr, hr, acc, easem, gsem, ssem, isem):
    cid = lax.axis_index("c")
    sid = lax.axis_index("s")
    wid = cid * _NS + sid

    zero16 = jnp.zeros((16,), jnp.float32)

    def _zrow(i, c):
        for j in range(8):
            hr[i, pl.ds(j * 16, 16)] = zero16
        return c

    lax.fori_loop(0, _ZR, _zrow, 0)
    for t in range(_NPT // _ZR):
        pltpu.sync_copy(hr.at[pl.ds(0, _ZR)],
                        acc.at[pl.ds(sid * _NPT + t * _ZR, _ZR)])
    plsc.subcore_barrier()

    gbase = l * _E + wid * _EPT

    def _fire(base, k2, p):
        b = k2 % 2
        pltpu.async_copy(ea_hbm.at[pl.ds(base + k2 * _CH, _CH)],
                         msgr.at[pl.ds(b * _CH, _CH)], easem.at[b])
        pltpu.async_copy(h_hbm.at[idxb.at[p, 2 * k2]],
                         hr.at[pl.ds(b * _CH, _CH)], gsem.at[b])

    def _wait_streams(base, k2, p):
        b = k2 % 2
        pltpu.make_async_copy(ea_hbm.at[pl.ds(base + k2 * _CH, _CH)],
                              msgr.at[pl.ds(b * _CH, _CH)],
                              easem.at[b]).wait()
        pltpu.make_async_copy(h_hbm.at[idxb.at[p, 2 * k2]],
                              hr.at[pl.ds(b * _CH, _CH)], gsem.at[b]).wait()

    def _fire_scatter(k2, p):
        b = k2 % 2
        pltpu.async_copy(hr.at[pl.ds(b * _CH, _CH)],
                         acc.at[idxb.at[p, 2 * k2 + 1]], ssem.at[b],
                         add=True)

    def _wait_scatter(b):
        pltpu.make_async_copy(hr.at[pl.ds(b * _CH, _CH)],
                              acc.at[idxb.at[0, 1]], ssem.at[b]).wait()

    def _relu(b):
        def _body(e, c):
            r = b * _CH + e
            for j in range(8):
                sl = pl.ds(j * 16, 16)
                hr[r, sl] = jnp.maximum(msgr[r, sl] + hr[r, sl], 0.0)
            return c
        lax.fori_loop(0, _CH, _body, 0)

    def _chunk_steps(base, k2, p, pn, bb):
        b = k2 % 2
        bn = (k2 + 1) % 2
        if k2 == 0:
            if bb is None:
                _wait_scatter(bn)
            else:
                @pl.when(bb > 0)
                def _():
                    _wait_scatter(bn)
        else:
            _wait_scatter(bn)
        if bb is not None and k2 == 1:
            pltpu.async_copy(idx_hbm.at[wid, bb + 1], idxb.at[pn], isem)
        if k2 < _BLK - 1 and not (bb is None and k2 == _TAIL - 1):
            _fire(base, k2 + 1, p)
        elif bb is not None:
            pltpu.make_async_copy(idx_hbm.at[wid, bb + 1], idxb.at[pn],
                                  isem).wait()
            _fire(base + _BLK * _CH, 0, pn)
        _wait_streams(base, k2, p)
        _relu(b)
        _fire_scatter(k2, p)

    pltpu.sync_copy(idx_hbm.at[wid, 0], idxb.at[0])
    _fire(gbase, 0, 0)

    def _block(bb, c):
        p = lax.rem(bb, 2)
        pn = lax.rem(bb + 1, 2)
        base = gbase + bb * (_BLK * _CH)
        for k2 in range(_BLK):
            _chunk_steps(base, k2, p, pn, bb)
        return c

    lax.fori_loop(0, _NBF, _block, 0)

    tbase = gbase + _NBF * _BLK * _CH
    for k2 in range(_TAIL):
        _chunk_steps(tbase, k2, 0, None, None)
    _wait_scatter((_TAIL - 1) % 2)
    plsc.subcore_barrier()

    for t in range(_NPT // _ZR):
        sl = pl.ds(sid * _NPT + t * _ZR, _ZR)
        pltpu.sync_copy(acc.at[sl], out_hbm.at[cid, sl])


@functools.cache
def _make_msgpass(l):
    return functools.partial(
        pl.kernel,
        mesh=plsc.VectorSubcoreMesh(core_axis_name="c", subcore_axis_name="s",
                                    num_cores=_NC, num_subcores=_NS),
        out_type=jax.ShapeDtypeStruct((_NC, _NP, _D), jnp.float32),
        scratch_types=[
            pltpu.VMEM((2, 2 * _BLK, _CH), jnp.int32),
            pltpu.VMEM((2 * _CH, _D), jnp.float32),
            pltpu.VMEM((2 * _CH, _D), jnp.float32),
            pltpu.VMEM_SHARED((_NP, _D), jnp.float32),
            pltpu.SemaphoreType.DMA((2,)),
            pltpu.SemaphoreType.DMA((2,)),
            pltpu.SemaphoreType.DMA((2,)),
            pltpu.SemaphoreType.DMA,
        ],
    )(functools.partial(_msgpass_body, l))


def _mlp_body(h_ref, p_ref, w1_ref, b1_ref, g_ref, be_ref, w2_ref, b2_ref,
              out_ref):
    z = h_ref[...] + p_ref[0, :_N] + p_ref[1, :_N]
    z = jnp.dot(z, w1_ref[...], preferred_element_type=jnp.float32) + b1_ref[...]
    mu = jnp.mean(z, axis=0, keepdims=True)
    var = jnp.mean((z - mu) * (z - mu), axis=0, keepdims=True)
    z = (z - mu) / jnp.sqrt(var + 1e-5) * g_ref[...] + be_ref[...]
    z = jnp.maximum(z, 0.0)
    z = jnp.dot(z, w2_ref[...], preferred_element_type=jnp.float32) + b2_ref[...]
    out_ref[...] = jnp.maximum(z, 0.0)


_mlp = pl.pallas_call(
    _mlp_body,
    out_shape=jax.ShapeDtypeStruct((_N, _D), jnp.float32),
)


def _pool_body(h_ref, b_ref, wo1_ref, bo1_ref, wo2_ref, bo2_ref, out_ref):
    bids = b_ref[...]
    gi = lax.broadcasted_iota(jnp.int32, (_N, _G), 1)
    mask = (bids == gi).astype(jnp.float32)
    dn = (((0,), (0,)), ((), ()))
    sums = lax.dot_general(mask, h_ref[...], dn,
                           preferred_element_type=jnp.float32)
    ones = jnp.ones((_N, 1), jnp.float32)
    cnt = lax.dot_general(mask, ones, dn,
                          preferred_element_type=jnp.float32)
    pooled = sums / jnp.maximum(cnt, 1.0)
    t = jnp.maximum(
        jnp.dot(pooled, wo1_ref[...], preferred_element_type=jnp.float32)
        + bo1_ref[...], 0.0)
    out_ref[...] = (
        jnp.dot(t, wo2_ref[...], preferred_element_type=jnp.float32)
        + bo2_ref[...])


_pool = pl.pallas_call(
    _pool_body,
    out_shape=jax.ShapeDtypeStruct((_G, _D), jnp.float32),
)


def kernel(x, edge_index, edge_attr, batch, edge_W, edge_b, W1, b1, gamma,
           beta, W2, b2, Wo1, bo1, Wo2, bo2):
    src = edge_index[0].reshape(_NW, _NCH, 1, _CH)
    dst = edge_index[1].reshape(_NW, _NCH, 1, _CH)
    idx = jnp.concatenate([src, dst], axis=2)
    idx = jnp.pad(idx, ((0, 0), (0, _BLK - _TAIL), (0, 0), (0, 0)))
    idx = idx.reshape(_NW, _NBF + 1, 2 * _BLK, _CH)

    ea_all = _embed(edge_attr, edge_W, edge_b.reshape(_L, 1, _D))
    ea_flat = ea_all.reshape(_L * _E, _D)

    h = x
    for l in range(_L):
        parts = _make_msgpass(l)(h, ea_flat, idx)
        h = _mlp(h, parts, W1[l], b1[l].reshape(1, _D),
                 gamma[l].reshape(1, _D), beta[l].reshape(1, _D),
                 W2[l], b2[l].reshape(1, _D))

    return _pool(h, batch.reshape(_N, 1), Wo1, bo1.reshape(1, _D),
                 Wo2, bo2.reshape(1, _D))


# speedup vs baseline: 1.1386x; 1.0072x over previous
import functools

import jax
import jax.numpy as jnp
from jax import lax
from jax.experimental import pallas as pl
from jax.experimental.pallas import tpu as pltpu
from jax.experimental.pallas import tpu_sc as plsc

_N = 10000
_E = 320000
_D = 128
_DE = 16
_L = 3
_G = 64

_NC = 2
_NS = 16
_NW = _NC * _NS
_EPT = _E // _NW
_CH = 80
_NCH = _EPT // _CH
_BLK = 10
_NBF = 12
_TAIL = _NCH - _NBF * _BLK
_NP = 10240
_NPT = _NP // _NS
_ZR = 160

_BE = 3200


def _embed_body(attr_ref, w_ref, b_ref, out_ref):
    a = attr_ref[...]
    for l in range(_L):
        out_ref[l] = (
            jnp.dot(a, w_ref[l], preferred_element_type=jnp.float32) + b_ref[l]
        )


_embed = pl.pallas_call(
    _embed_body,
    grid=(_E // _BE,),
    in_specs=[
        pl.BlockSpec((_BE, _DE), lambda i: (i, 0)),
        pl.BlockSpec((_L, _DE, _D), lambda i: (0, 0, 0)),
        pl.BlockSpec((_L, 1, _D), lambda i: (0, 0, 0)),
    ],
    out_specs=pl.BlockSpec((_L, _BE, _D), lambda i: (0, i, 0)),
    out_shape=jax.ShapeDtypeStruct((_L, _E, _D), jnp.float32),
)


def _msgpass_body(l, h_hbm, ea_hbm, idx_hbm, out_hbm,
                  idxb, msgr, hr, acc, easem, gsem, ssem, isem):
    cid = lax.axis_index("c")
    sid = lax.axis_index("s")
    wid = cid * _NS + sid

    zero16 = jnp.zeros((16,), jnp.float32)

    def _zrow(i, c):
        for j in range(8):
            hr[i, pl.ds(j * 16, 16)] = zero16
        return c

    lax.fori_loop(0, _ZR, _zrow, 0)
    for t in range(_NPT // _ZR):
        pltpu.sync_copy(hr.at[pl.ds(0, _ZR)],
                        acc.at[pl.ds(sid * _NPT + t * _ZR, _ZR)])
    plsc.subcore_barrier()

    gbase = l * _E + wid * _EPT

    def _fire(base, k2, p):
        b = k2 % 2
        pltpu.async_copy(ea_hbm.at[pl.ds(base + k2 * _CH, _CH)],
                         msgr.at[pl.ds(b * _CH, _CH)], easem.at[b])
        pltpu.async_copy(h_hbm.at[idxb.at[p, 2 * k2]],
                         hr.at[pl.ds(b * _CH, _CH)], gsem.at[b])

    def _wait_streams(base, k2, p):
        b = k2 % 2
        pltpu.make_async_copy(ea_hbm.at[pl.ds(base + k2 * _CH, _CH)],
                              msgr.at[pl.ds(b * _CH, _CH)],
                              easem.at[b]).wait()
        pltpu.make_async_copy(h_hbm.at[idxb.at[p, 2 * k2]],
                              hr.at[pl.ds(b * _CH, _CH)], gsem.at[b]).wait()

    def _fire_scatter(k2, p):
        b = k2 % 2
        pltpu.async_copy(hr.at[pl.ds(b * _CH, _CH)],
                         acc.at[idxb.at[p, 2 * k2 + 1]], ssem.at[b],
                         add=True)

    def _wait_scatter(b):
        pltpu.make_async_copy(hr.at[pl.ds(b * _CH, _CH)],
                              acc.at[idxb.at[0, 1]], ssem.at[b]).wait()

    def _relu(b):
        def _body(e, c):
            r = b * _CH + 2 * e
            for dr in range(2):
                for j in range(8):
                    sl = pl.ds(j * 16, 16)
                    hr[r + dr, sl] = jnp.maximum(
                        msgr[r + dr, sl] + hr[r + dr, sl], 0.0)
            return c
        lax.fori_loop(0, _CH // 2, _body, 0)

    def _chunk_steps(base, k2, p, pn, bb):
        b = k2 % 2
        bn = (k2 + 1) % 2
        if k2 == 0:
            if bb is None:
                _wait_scatter(bn)
            else:
                @pl.when(bb > 0)
                def _():
                    _wait_scatter(bn)
        else:
            _wait_scatter(bn)
        if bb is not None and k2 == 1:
            pltpu.async_copy(idx_hbm.at[wid, bb + 1], idxb.at[pn], isem)
        if k2 < _BLK - 1 and not (bb is None and k2 == _TAIL - 1):
            _fire(base, k2 + 1, p)
        elif bb is not None:
            pltpu.make_async_copy(idx_hbm.at[wid, bb + 1], idxb.at[pn],
                                  isem).wait()
            _fire(base + _BLK * _CH, 0, pn)
        _wait_streams(base, k2, p)
        _relu(b)
        _fire_scatter(k2, p)

    pltpu.sync_copy(idx_hbm.at[wid, 0], idxb.at[0])
    _fire(gbase, 0, 0)

    def _block(bb, c):
        p = lax.rem(bb, 2)
        pn = lax.rem(bb + 1, 2)
        base = gbase + bb * (_BLK * _CH)
        for k2 in range(_BLK):
            _chunk_steps(base, k2, p, pn, bb)
        return c

    lax.fori_loop(0, _NBF, _block, 0)

    tbase = gbase + _NBF * _BLK * _CH
    for k2 in range(_TAIL):
        _chunk_steps(tbase, k2, 0, None, None)
    _wait_scatter((_TAIL - 1) % 2)
    plsc.subcore_barrier()

    for t in range(_NPT // _ZR):
        sl = pl.ds(sid * _NPT + t * _ZR, _ZR)
        pltpu.sync_copy(acc.at[sl], out_hbm.at[cid, sl])


@functools.cache
def _make_msgpass(l):
    return functools.partial(
        pl.kernel,
        mesh=plsc.VectorSubcoreMesh(core_axis_name="c", subcore_axis_name="s",
                                    num_cores=_NC, num_subcores=_NS),
        out_type=jax.ShapeDtypeStruct((_NC, _NP, _D), jnp.float32),
        scratch_types=[
            pltpu.VMEM((2, 2 * _BLK, _CH), jnp.int32),
            pltpu.VMEM((2 * _CH, _D), jnp.float32),
            pltpu.VMEM((2 * _CH, _D), jnp.float32),
            pltpu.VMEM_SHARED((_NP, _D), jnp.float32),
            pltpu.SemaphoreType.DMA((2,)),
            pltpu.SemaphoreType.DMA((2,)),
            pltpu.SemaphoreType.DMA((2,)),
            pltpu.SemaphoreType.DMA,
        ],
    )(functools.partial(_msgpass_body, l))


def _mlp_body(h_ref, p_ref, w1_ref, b1_ref, g_ref, be_ref, w2_ref, b2_ref,
              out_ref):
    z = h_ref[...] + p_ref[0, :_N] + p_ref[1, :_N]
    z = jnp.dot(z, w1_ref[...], preferred_element_type=jnp.float32) + b1_ref[...]
    mu = jnp.mean(z, axis=0, keepdims=True)
    var = jnp.mean((z - mu) * (z - mu), axis=0, keepdims=True)
    z = (z - mu) / jnp.sqrt(var + 1e-5) * g_ref[...] + be_ref[...]
    z = jnp.maximum(z, 0.0)
    z = jnp.dot(z, w2_ref[...], preferred_element_type=jnp.float32) + b2_ref[...]
    out_ref[...] = jnp.maximum(z, 0.0)


_mlp = pl.pallas_call(
    _mlp_body,
    out_shape=jax.ShapeDtypeStruct((_N, _D), jnp.float32),
)


def _pool_body(h_ref, b_ref, wo1_ref, bo1_ref, wo2_ref, bo2_ref, out_ref):
    bids = b_ref[...]
    gi = lax.broadcasted_iota(jnp.int32, (_N, _G), 1)
    mask = (bids == gi).astype(jnp.float32)
    dn = (((0,), (0,)), ((), ()))
    sums = lax.dot_general(mask, h_ref[...], dn,
                           preferred_element_type=jnp.float32)
    ones = jnp.ones((_N, 1), jnp.float32)
    cnt = lax.dot_general(mask, ones, dn,
                          preferred_element_type=jnp.float32)
    pooled = sums / jnp.maximum(cnt, 1.0)
    t = jnp.maximum(
        jnp.dot(pooled, wo1_ref[...], preferred_element_type=jnp.float32)
        + bo1_ref[...], 0.0)
    out_ref[...] = (
        jnp.dot(t, wo2_ref[...], preferred_element_type=jnp.float32)
        + bo2_ref[...])


_pool = pl.pallas_call(
    _pool_body,
    out_shape=jax.ShapeDtypeStruct((_G, _D), jnp.float32),
)


def _mlp_pool_body(h_ref, p_ref, w1_ref, b1_ref, g_ref, be_ref, w2_ref,
                   b2_ref, b_ref, wo1_ref, bo1_ref, wo2_ref, bo2_ref,
                   out_ref):
    z = h_ref[...] + p_ref[0, :_N] + p_ref[1, :_N]
    z = jnp.dot(z, w1_ref[...], preferred_element_type=jnp.float32) + b1_ref[...]
    mu = jnp.mean(z, axis=0, keepdims=True)
    var = jnp.mean((z - mu) * (z - mu), axis=0, keepdims=True)
    z = (z - mu) / jnp.sqrt(var + 1e-5) * g_ref[...] + be_ref[...]
    z = jnp.maximum(z, 0.0)
    z = jnp.dot(z, w2_ref[...], preferred_element_type=jnp.float32) + b2_ref[...]
    h = jnp.maximum(z, 0.0)
    bids = b_ref[...]
    gi = lax.broadcasted_iota(jnp.int32, (_N, _G), 1)
    mask = (bids == gi).astype(jnp.float32)
    dn = (((0,), (0,)), ((), ()))
    sums = lax.dot_general(mask, h, dn, preferred_element_type=jnp.float32)
    ones = jnp.ones((_N, 1), jnp.float32)
    cnt = lax.dot_general(mask, ones, dn, preferred_element_type=jnp.float32)
    pooled = sums / jnp.maximum(cnt, 1.0)
    t = jnp.maximum(
        jnp.dot(pooled, wo1_ref[...], preferred_element_type=jnp.float32)
        + bo1_ref[...], 0.0)
    out_ref[...] = (
        jnp.dot(t, wo2_ref[...], preferred_element_type=jnp.float32)
        + bo2_ref[...])


_mlp_pool = pl.pallas_call(
    _mlp_pool_body,
    out_shape=jax.ShapeDtypeStruct((_G, _D), jnp.float32),
)


def kernel(x, edge_index, edge_attr, batch, edge_W, edge_b, W1, b1, gamma,
           beta, W2, b2, Wo1, bo1, Wo2, bo2):
    src = edge_index[0].reshape(_NW, _NCH, 1, _CH)
    dst = edge_index[1].reshape(_NW, _NCH, 1, _CH)
    idx = jnp.concatenate([src, dst], axis=2)
    idx = jnp.pad(idx, ((0, 0), (0, _BLK - _TAIL), (0, 0), (0, 0)))
    idx = idx.reshape(_NW, _NBF + 1, 2 * _BLK, _CH)

    ea_all = _embed(edge_attr, edge_W, edge_b.reshape(_L, 1, _D))
    ea_flat = ea_all.reshape(_L * _E, _D)

    h = x
    for l in range(_L - 1):
        parts = _make_msgpass(l)(h, ea_flat, idx)
        h = _mlp(h, parts, W1[l], b1[l].reshape(1, _D),
                 gamma[l].reshape(1, _D), beta[l].reshape(1, _D),
                 W2[l], b2[l].reshape(1, _D))

    l = _L - 1
    parts = _make_msgpass(l)(h, ea_flat, idx)
    return _mlp_pool(h, parts, W1[l], b1[l].reshape(1, _D),
                     gamma[l].reshape(1, _D), beta[l].reshape(1, _D),
                     W2[l], b2[l].reshape(1, _D), batch.reshape(_N, 1),
                     Wo1, bo1.reshape(1, _D), Wo2, bo2.reshape(1, _D))
